# baseline (speedup 1.0000x reference)
import jax
import jax.numpy as jnp
from jax.experimental import pallas as pl

N = 10000
E = 320000
HID = 128
H = 8


def _gat(h, src, dst, fc_W, attn_l, attn_r, res_W, gat_b):
    Nn = h.shape[0]
    feat = (h @ fc_W).reshape(Nn, H, HID)
    el = jnp.sum(feat * attn_l[None, :, :], axis=-1)
    er = jnp.sum(feat * attn_r[None, :, :], axis=-1)
    e = jax.nn.leaky_relu(el[src] + er[dst], negative_slope=0.2)
    emax = jax.ops.segment_max(e, dst, num_segments=Nn)
    emax = jnp.where(jnp.isfinite(emax), emax, 0.0)
    ee = jnp.exp(e - emax[dst])
    denom = jax.ops.segment_sum(ee, dst, num_segments=Nn)
    alpha = ee / denom[dst]
    msg = feat[src] * alpha[:, :, None]
    rst = jax.ops.segment_sum(msg, dst, num_segments=Nn)
    rst = rst + (h @ res_W).reshape(Nn, H, HID)
    rst = rst + gat_b.reshape(1, H, HID)
    rst = jax.nn.leaky_relu(rst, negative_slope=0.01)
    return rst.reshape(Nn, H * HID)


def _tail_kernel(pooled_ref, w_ref, b_ref, o_ref):
    o_ref[...] = pooled_ref[...] @ w_ref[...] + b_ref[...]


def kernel(x, edge_index, enc_W, enc_b, fc_W0, attn_l0, attn_r0, res_W0, gat_b0, down_W0, down_b0, fc_W1, attn_l1, attn_r1, res_W1, gat_b1, down_W1, down_b1, gate_W, gate_b, bn_gamma, bn_beta, cls_W, cls_b):
    src = edge_index[0]
    dst = edge_index[1]
    he = x @ enc_W + enc_b
    h = _gat(he, src, dst, fc_W0, attn_l0, attn_r0, res_W0, gat_b0) @ down_W0 + down_b0
    h = _gat(h, src, dst, fc_W1, attn_l1, attn_r1, res_W1, gat_b1) @ down_W1 + down_b1
    hg = jnp.concatenate([h, he], axis=1)
    mean = jnp.mean(hg, axis=0)
    var = jnp.var(hg, axis=0)
    hg = bn_gamma * (hg - mean) / jnp.sqrt(var + 1e-5) + bn_beta
    gate = jax.nn.softmax(hg @ gate_W + gate_b, axis=0)
    pooled = jnp.sum(gate * hg, axis=0, keepdims=True)
    out = pl.pallas_call(
        _tail_kernel,
        out_shape=jax.ShapeDtypeStruct((1, cls_W.shape[1]), jnp.float32),
    )(pooled, cls_W, cls_b.reshape(1, -1))
    return out


# speedup vs baseline: 18.6827x; 18.6827x over previous
import functools

import jax
import jax.numpy as jnp
from jax import lax
from jax.experimental import pallas as pl
from jax.experimental.pallas import tpu as pltpu
from jax.experimental.pallas import tpu_sc as plsc

NN = 10000
EE = 320000
DIN = 128
HID = 128
NH = 8
WB = 64
NBK = 160
NPAD = NBK * WB
NWORK = 32
ROUNDS = NBK // NWORK
KA = 128
EPAD = EE + 2 * KA
ZROW = NH * HID
ZWORDS = WB * ZROW
HI = jax.lax.Precision.HIGHEST


def _sc_edge_body(src_hbm, dst_hbm, off_hbm, elr_hbm, erf_hbm, h_hbm,
                  z_hbm,
                  offv, srcbuf, dstbuf, elbuf, erv, hbuf, albuf,
                  den, rden, zacc):
    wid = lax.axis_index("s") * 2 + lax.axis_index("c")
    iota = lax.iota(jnp.int32, 16)

    pltpu.sync_copy(off_hbm, offv)

    def splat(x):
        return jnp.full((16,), x, jnp.int32)

    def read_off(b):
        return jnp.max(plsc.load_gather(offv, [splat(b)]))

    def round_body(r, _carry):
        b = wid + NWORK * r
        base = b * WB
        off_b = read_off(b)
        off_b1 = read_off(b + 1)
        a_start = jnp.bitwise_and(off_b, jnp.int32(-8))
        nsteps = (off_b1 - a_start + (KA - 1)) // KA

        pltpu.sync_copy(
            erf_hbm.at[pl.ds(pl.multiple_of(b * (WB * 16), 8), WB * 16)], erv)

        def zero_z(i, c):
            plsc.store_scatter(zacc, [i * 16 + iota],
                               jnp.zeros((16,), jnp.float32))
            return c
        lax.fori_loop(0, ZWORDS // 16, zero_z, 0, unroll=8)
        for i in range(WB * NH // 16):
            plsc.store_scatter(den, [i * 16 + iota],
                               jnp.zeros((16,), jnp.float32))

        def stageA(step, c):
            gbase = pl.multiple_of(a_start + step * KA, 8)
            pltpu.sync_copy(src_hbm.at[pl.ds(gbase, KA)], srcbuf)
            pltpu.sync_copy(dst_hbm.at[pl.ds(gbase, KA)], dstbuf)
            pltpu.sync_copy(elr_hbm.at[srcbuf], elbuf)
            for j in range(KA // 16):
                cidx = j * 16 + iota
                gpos = gbase + cidx
                dstv = dstbuf[pl.ds(j * 16, 16)]
                valid = jnp.logical_and(gpos >= off_b, gpos < off_b1)
                dloc = jnp.clip(dstv - base, 0, WB - 1)
                prev = dstv.at[jnp.maximum(iota - 1, 0)].get(
                    mode="promise_in_bounds")
                nxt = dstv.at[jnp.minimum(iota + 1, 15)].get(
                    mode="promise_in_bounds")
                start_run = jnp.logical_or(iota == 0, dstv != prev)
                last_run = jnp.logical_or(iota == 15, dstv != nxt)
                run_start = plsc.cummax(jnp.where(start_run, iota, 0))
                prev_idx = jnp.maximum(run_start - 1, 0)
                for h in range(NH):
                    elh = plsc.load_gather(elbuf, [cidx, splat(h)])
                    erh = plsc.load_gather(erv, [dloc * 16 + (8 + h)])
                    e = elh + erh
                    e = jnp.where(e >= 0, e, 0.2 * e)
                    ee = jnp.where(valid, jnp.exp(e), 0.0)
                    cs = plsc.cumsum(ee)
                    pcs = cs.at[prev_idx].get(mode="promise_in_bounds")
                    tot = cs - jnp.where(run_start > 0, pcs, 0.0)
                    plsc.addupdate_scatter(den, [dloc * NH + h], tot,
                                           mask=last_run)
            return c
        lax.fori_loop(0, nsteps, stageA, 0)

        for i in range(WB * NH // 16):
            d16 = den[pl.ds(i * 16, 16)]
            plsc.store_scatter(rden, [i * 16 + iota],
                               jnp.where(d16 > 0, 1.0 / d16, 0.0))

        def stageB(step, c):
            gbase = pl.multiple_of(a_start + step * KA, 8)
            pltpu.sync_copy(src_hbm.at[pl.ds(gbase, KA)], srcbuf)
            pltpu.sync_copy(dst_hbm.at[pl.ds(gbase, KA)], dstbuf)
            pltpu.sync_copy(elr_hbm.at[srcbuf], elbuf)
            pltpu.sync_copy(h_hbm.at[srcbuf], hbuf)
            for j in range(KA // 16):
                cidx = j * 16 + iota
                gpos = gbase + cidx
                dstv = dstbuf[pl.ds(j * 16, 16)]
                valid = jnp.logical_and(gpos >= off_b, gpos < off_b1)
                dloc = jnp.clip(dstv - base, 0, WB - 1)
                for h in range(NH):
                    elh = plsc.load_gather(elbuf, [cidx, splat(h)])
                    erh = plsc.load_gather(erv, [dloc * 16 + (8 + h)])
                    e = elh + erh
                    e = jnp.where(e >= 0, e, 0.2 * e)
                    ee = jnp.where(valid, jnp.exp(e), 0.0)
                    rd = plsc.load_gather(rden, [dloc * NH + h])
                    plsc.store_scatter(albuf, [cidx * NH + h], ee * rd)

            def agg_edge(ei, c2):
                dl16 = plsc.load_gather(dstbuf, [splat(ei)])
                dl16 = jnp.clip(dl16 - base, 0, WB - 1)
                zb = dl16 * ZROW
                hv = [plsc.load_gather(hbuf, [splat(ei), v * 16 + iota])
                      for v in range(HID // 16)]
                for h in range(NH):
                    a16 = plsc.load_gather(albuf, [splat(ei * NH + h)])
                    for v in range(HID // 16):
                        plsc.addupdate_scatter(
                            zacc, [zb + h * HID + v * 16 + iota], a16 * hv[v])
                return c2
            lax.fori_loop(0, KA, agg_edge, 0)
            return c
        lax.fori_loop(0, nsteps, stageB, 0)

        pltpu.sync_copy(zacc,
                        z_hbm.at[pl.ds(pl.multiple_of(b * ZWORDS, 8), ZWORDS)])
        return _carry

    lax.fori_loop(0, ROUNDS, round_body, 0)


def _sc_edge(src_p, dst_p, off, elr_pad, er_flat, h):
    mesh = plsc.VectorSubcoreMesh(core_axis_name="c", subcore_axis_name="s")
    fn = pl.kernel(
        _sc_edge_body,
        out_type=jax.ShapeDtypeStruct((NPAD * ZROW,), jnp.float32),
        mesh=mesh,
        compiler_params=pltpu.CompilerParams(needs_layout_passes=False),
        scratch_types=[
            pltpu.VMEM((176,), jnp.int32),
            pltpu.VMEM((KA,), jnp.int32),
            pltpu.VMEM((KA,), jnp.int32),
            pltpu.VMEM((KA, HID), jnp.float32),
            pltpu.VMEM((WB * 16,), jnp.float32),
            pltpu.VMEM((KA, HID), jnp.float32),
            pltpu.VMEM((KA * NH,), jnp.float32),
            pltpu.VMEM((WB * NH,), jnp.float32),
            pltpu.VMEM((WB * NH,), jnp.float32),
            pltpu.VMEM((ZWORDS,), jnp.float32),
        ],
    )
    return fn(src_p, dst_p, off, elr_pad, er_flat, h)


def _fold_attn(fc_W, attn_l, attn_r):
    cols = []
    for h in range(NH):
        cols.append(jnp.dot(fc_W[:, h * HID:(h + 1) * HID], attn_l[h],
                            precision=HI))
    for h in range(NH):
        cols.append(jnp.dot(fc_W[:, h * HID:(h + 1) * HID], attn_r[h],
                            precision=HI))
    A = jnp.stack(cols, axis=1)
    return jnp.concatenate([A, jnp.zeros((HID, HID - 2 * NH), jnp.float32)],
                           axis=1)


def _tc_encode_body(x_ref, encW_ref, encb_ref, fcW_ref, al_ref, ar_ref,
                    he_ref, elr_ref):
    he = jnp.dot(x_ref[...], encW_ref[...], precision=HI) + encb_ref[...]
    he_ref[...] = he
    A = _fold_attn(fcW_ref[...], al_ref[...], ar_ref[...])
    elr_ref[...] = jnp.dot(he, A, precision=HI)


def _tc_encode(x, enc_W, enc_b, fc_W0, attn_l0, attn_r0):
    return pl.pallas_call(
        _tc_encode_body,
        out_shape=(jax.ShapeDtypeStruct((NN, HID), jnp.float32),
                   jax.ShapeDtypeStruct((NN, HID), jnp.float32)),
    )(x, enc_W, enc_b.reshape(1, HID), fc_W0, attn_l0, attn_r0)


def _tc_post_body(z_ref, hin_ref, fcW_ref, resW_ref, gatb_ref,
                  downW_ref, downb_ref, fcWn_ref, aln_ref, arn_ref,
                  hout_ref, elr_ref, *, with_elr):
    z = z_ref[...]
    hin = hin_ref[...]
    fcW = fcW_ref[...]
    parts = []
    for h in range(NH):
        parts.append(jnp.dot(z[:, h * HID:(h + 1) * HID],
                             fcW[:, h * HID:(h + 1) * HID], precision=HI))
    rst = jnp.concatenate(parts, axis=1)
    rst = rst + jnp.dot(hin, resW_ref[...], precision=HI)
    rst = rst + gatb_ref[...]
    rst = jnp.where(rst >= 0, rst, 0.01 * rst)
    hout = jnp.dot(rst, downW_ref[...], precision=HI) + downb_ref[...]
    hout_ref[...] = hout
    if with_elr:
        A = _fold_attn(fcWn_ref[...], aln_ref[...], arn_ref[...])
        elr_ref[...] = jnp.dot(hout, A, precision=HI)
    else:
        elr_ref[...] = jnp.zeros_like(elr_ref)


def _tc_post(z, hin, fc_W, res_W, gat_b, down_W, down_b,
             fc_Wn, attn_ln, attn_rn, with_elr):
    blk = 2000
    grid = NN // blk
    return pl.pallas_call(
        functools.partial(_tc_post_body, with_elr=with_elr),
        grid=(grid,),
        in_specs=[
            pl.BlockSpec((blk, ZROW), lambda i: (i, 0)),
            pl.BlockSpec((blk, HID), lambda i: (i, 0)),
            pl.BlockSpec((HID, ZROW), lambda i: (0, 0)),
            pl.BlockSpec((HID, ZROW), lambda i: (0, 0)),
            pl.BlockSpec((1, ZROW), lambda i: (0, 0)),
            pl.BlockSpec((ZROW, HID), lambda i: (0, 0)),
            pl.BlockSpec((1, HID), lambda i: (0, 0)),
            pl.BlockSpec((HID, ZROW), lambda i: (0, 0)),
            pl.BlockSpec((NH, HID), lambda i: (0, 0)),
            pl.BlockSpec((NH, HID), lambda i: (0, 0)),
        ],
        out_specs=(pl.BlockSpec((blk, HID), lambda i: (i, 0)),
                   pl.BlockSpec((blk, HID), lambda i: (i, 0))),
        out_shape=(jax.ShapeDtypeStruct((NN, HID), jnp.float32),
                   jax.ShapeDtypeStruct((NN, HID), jnp.float32)),
    )(z, hin, fc_W, res_W, gat_b.reshape(1, ZROW), down_W,
      down_b.reshape(1, HID), fc_Wn, attn_ln, attn_rn)


def _tc_head_body(h_ref, he_ref, gW_ref, gb_ref, gam_ref, bet_ref,
                  cW_ref, cb_ref, out_ref):
    hg = jnp.concatenate([h_ref[...], he_ref[...]], axis=1)
    mean = jnp.mean(hg, axis=0, keepdims=True)
    var = jnp.mean((hg - mean) ** 2, axis=0, keepdims=True)
    hg = gam_ref[...] * (hg - mean) / jnp.sqrt(var + 1e-5) + bet_ref[...]
    logit = jnp.dot(hg, gW_ref[...], precision=HI) + gb_ref[...]
    m = jnp.max(logit, axis=0, keepdims=True)
    eg = jnp.exp(logit - m)
    gate = eg / jnp.sum(eg, axis=0, keepdims=True)
    pooled = jnp.sum(gate * hg, axis=0, keepdims=True)
    out_ref[...] = jnp.dot(pooled, cW_ref[...], precision=HI) + cb_ref[...]


def _tc_head(h, he, gate_W, gate_b, bn_gamma, bn_beta, cls_W, cls_b):
    nc = cls_W.shape[1]
    return pl.pallas_call(
        _tc_head_body,
        out_shape=jax.ShapeDtypeStruct((1, nc), jnp.float32),
    )(h, he, gate_W, gate_b.reshape(1, 1), bn_gamma.reshape(1, 2 * HID),
      bn_beta.reshape(1, 2 * HID), cls_W, cls_b.reshape(1, nc))


def _er_flat(elr_pad):
    er = elr_pad[:, :16]
    er = jnp.concatenate([er, jnp.zeros((NPAD - NN, 16), jnp.float32)])
    return er.reshape(-1)


def kernel(x, edge_index, enc_W, enc_b, fc_W0, attn_l0, attn_r0, res_W0,
           gat_b0, down_W0, down_b0, fc_W1, attn_l1, attn_r1, res_W1,
           gat_b1, down_W1, down_b1, gate_W, gate_b, bn_gamma, bn_beta,
           cls_W, cls_b):
    src = edge_index[0]
    dst = edge_index[1]
    dst_s, src_s = lax.sort((dst, src), num_keys=1)
    off = jnp.searchsorted(dst_s, jnp.arange(NBK + 1, dtype=jnp.int32) * WB,
                           method="scan_unrolled").astype(jnp.int32)
    off = jnp.concatenate([off, jnp.zeros((15,), jnp.int32)])
    pad = jnp.zeros((EPAD - EE,), jnp.int32)
    src_p = jnp.concatenate([src_s, pad])
    dst_p = jnp.concatenate([dst_s, pad])

    he, elr0 = _tc_encode(x, enc_W, enc_b, fc_W0, attn_l0, attn_r0)

    z0 = _sc_edge(src_p, dst_p, off, elr0, _er_flat(elr0), he)
    z0 = z0.reshape(NPAD, ZROW)[:NN]
    h1, elr1 = _tc_post(z0, he, fc_W0, res_W0, gat_b0, down_W0, down_b0,
                        fc_W1, attn_l1, attn_r1, with_elr=True)

    z1 = _sc_edge(src_p, dst_p, off, elr1, _er_flat(elr1), h1)
    z1 = z1.reshape(NPAD, ZROW)[:NN]
    h2, _ = _tc_post(z1, h1, fc_W1, res_W1, gat_b1, down_W1, down_b1,
                     fc_W1, attn_l1, attn_r1, with_elr=False)

    return _tc_head(h2, he, gate_W, gate_b, bn_gamma, bn_beta, cls_W, cls_b)


# speedup vs baseline: 21.8568x; 1.1699x over previous
import functools

import jax
import jax.numpy as jnp
from jax import lax
from jax.experimental import pallas as pl
from jax.experimental.pallas import tpu as pltpu
from jax.experimental.pallas import tpu_sc as plsc

NN = 10000
EE = 320000
DIN = 128
HID = 128
NH = 8
WB = 48
NBK = 224
NPAD = NBK * WB
NWORK = 32
ROUNDS = NBK // NWORK
KA = 128
EPAD = EE + 2 * KA
ZROW = NH * HID
ZWORDS = WB * ZROW
GW = 2 * HID
HI = jax.lax.Precision.HIGHEST


def _sc_edge_body(src_hbm, dst_hbm, off_hbm, hx_hbm, erf_hbm, z_hbm,
                  offv, srcA, dstA, srcB, dstB, gbufA, gbufB, erv,
                  eebuf, den, rden, zacc,
                  sem0, sem1, semGA, semGB):
    wid = lax.axis_index("s") * 2 + lax.axis_index("c")
    iota = lax.iota(jnp.int32, 16)

    pltpu.sync_copy(off_hbm, offv)

    def splat(x):
        return jnp.full((16,), x, jnp.int32)

    def read_off(b):
        return jnp.max(plsc.load_gather(offv, [splat(b)]))

    def round_body(r, _carry):
        b = wid + NWORK * r
        base = b * WB
        off_b = read_off(b)
        off_b1 = read_off(b + 1)
        a_start = jnp.bitwise_and(off_b, jnp.int32(-8))
        nsteps = (off_b1 - a_start + (KA - 1)) // KA

        pltpu.sync_copy(
            erf_hbm.at[pl.ds(pl.multiple_of(b * (WB * 16), 8), WB * 16)], erv)

        def zero_z(i, c):
            plsc.store_scatter(zacc, [i * 16 + iota],
                               jnp.zeros((16,), jnp.float32))
            return c
        lax.fori_loop(0, ZWORDS // 16, zero_z, 0, unroll=8)
        for i in range(WB * NH // 16):
            plsc.store_scatter(den, [i * 16 + iota],
                               jnp.zeros((16,), jnp.float32))

        def gbase_of(step):
            return pl.multiple_of(a_start + step * KA, 8)

        def load_sd(step, sbuf, dbuf):
            gb = gbase_of(step)
            c0 = pltpu.async_copy(src_hbm.at[pl.ds(gb, KA)], sbuf, sem0)
            c1 = pltpu.async_copy(dst_hbm.at[pl.ds(gb, KA)], dbuf, sem1)
            c0.wait()
            c1.wait()

        @pl.when(nsteps > 0)
        def _():
            load_sd(0, srcA, dstA)
            pltpu.async_copy(hx_hbm.at[srcA], gbufA, semGA)

        def pass_one(step, sbuf, dbuf, gbuf, semG, sbuf2, dbuf2, gbuf2,
                     semG2):
            gbase = gbase_of(step)

            @pl.when(step + 1 < nsteps)
            def _():
                load_sd(step + 1, sbuf2, dbuf2)
                pltpu.async_copy(hx_hbm.at[sbuf2], gbuf2, semG2)

            pltpu.make_async_copy(hx_hbm.at[sbuf], gbuf, semG).wait()

            for j in range(KA // 16):
                cidx = j * 16 + iota
                gpos = gbase + cidx
                dstv = dbuf[pl.ds(j * 16, 16)]
                valid = jnp.logical_and(gpos >= off_b, gpos < off_b1)
                dloc = jnp.clip(dstv - base, 0, WB - 1)
                prev = dstv.at[jnp.maximum(iota - 1, 0)].get(
                    mode="promise_in_bounds")
                nxt = dstv.at[jnp.minimum(iota + 1, 15)].get(
                    mode="promise_in_bounds")
                start_run = jnp.logical_or(iota == 0, dstv != prev)
                last_run = jnp.logical_or(iota == 15, dstv != nxt)
                run_start = plsc.cummax(jnp.where(start_run, iota, 0))
                prev_idx = jnp.maximum(run_start - 1, 0)
                for h in range(NH):
                    elh = plsc.load_gather(gbuf, [cidx, splat(HID + h)])
                    erh = plsc.load_gather(erv, [dloc * 16 + (8 + h)])
                    e = elh + erh
                    e = jnp.where(e >= 0, e, 0.2 * e)
                    ee = jnp.where(valid, jnp.exp(e), 0.0)
                    plsc.store_scatter(eebuf, [cidx * NH + h], ee)
                    cs = plsc.cumsum(ee)
                    pcs = cs.at[prev_idx].get(mode="promise_in_bounds")
                    tot = cs - jnp.where(run_start > 0, pcs, 0.0)
                    plsc.addupdate_scatter(den, [dloc * NH + h], tot,
                                           mask=last_run)

            def agg_edge(ei, c2):
                dl16 = plsc.load_gather(dbuf, [splat(ei)])
                dl16 = jnp.clip(dl16 - base, 0, WB - 1)
                zb = dl16 * ZROW
                hv = [plsc.load_gather(gbuf, [splat(ei), v * 16 + iota])
                      for v in range(HID // 16)]
                for h in range(NH):
                    a16 = plsc.load_gather(eebuf, [splat(ei * NH + h)])
                    for v in range(HID // 16):
                        plsc.addupdate_scatter(
                            zacc, [zb + h * HID + v * 16 + iota], a16 * hv[v])
                return c2
            lax.fori_loop(0, KA, agg_edge, 0)

        def one_pass(step, c):
            @pl.when(jnp.bitwise_and(step, 1) == 0)
            def _():
                pass_one(step, srcA, dstA, gbufA, semGA,
                         srcB, dstB, gbufB, semGB)

            @pl.when(jnp.bitwise_and(step, 1) == 1)
            def _():
                pass_one(step, srcB, dstB, gbufB, semGB,
                         srcA, dstA, gbufA, semGA)
            return c
        lax.fori_loop(0, nsteps, one_pass, 0)

        for i in range(WB * NH // 16):
            d16 = den[pl.ds(i * 16, 16)]
            plsc.store_scatter(rden, [i * 16 + iota],
                               jnp.where(d16 > 0, 1.0 / d16, 0.0))

        def scale_z(i, c):
            w16 = i * 16 + iota
            rd = plsc.load_gather(rden, [lax.shift_right_logical(w16, 7)])
            z16 = plsc.load_gather(zacc, [w16])
            plsc.store_scatter(zacc, [w16], z16 * rd)
            return c
        lax.fori_loop(0, ZWORDS // 16, scale_z, 0, unroll=8)

        pltpu.sync_copy(zacc,
                        z_hbm.at[pl.ds(pl.multiple_of(b * ZWORDS, 8), ZWORDS)])
        return _carry

    lax.fori_loop(0, ROUNDS, round_body, 0)


def _sc_edge(src_p, dst_p, off, hx, er_flat):
    mesh = plsc.VectorSubcoreMesh(core_axis_name="c", subcore_axis_name="s")
    fn = pl.kernel(
        _sc_edge_body,
        out_type=jax.ShapeDtypeStruct((NPAD * ZROW,), jnp.float32),
        mesh=mesh,
        compiler_params=pltpu.CompilerParams(needs_layout_passes=False),
        scratch_types=[
            pltpu.VMEM((240,), jnp.int32),
            pltpu.VMEM((KA,), jnp.int32),
            pltpu.VMEM((KA,), jnp.int32),
            pltpu.VMEM((KA,), jnp.int32),
            pltpu.VMEM((KA,), jnp.int32),
            pltpu.VMEM((KA, GW), jnp.float32),
            pltpu.VMEM((KA, GW), jnp.float32),
            pltpu.VMEM((WB * 16,), jnp.float32),
            pltpu.VMEM((KA * NH,), jnp.float32),
            pltpu.VMEM((WB * NH,), jnp.float32),
            pltpu.VMEM((WB * NH,), jnp.float32),
            pltpu.VMEM((ZWORDS,), jnp.float32),
            pltpu.SemaphoreType.DMA,
            pltpu.SemaphoreType.DMA,
            pltpu.SemaphoreType.DMA,
            pltpu.SemaphoreType.DMA,
        ],
    )
    return fn(src_p, dst_p, off, hx, er_flat)


def _fold_attn(fc_W, attn_l, attn_r):
    cols = []
    for h in range(NH):
        cols.append(jnp.dot(fc_W[:, h * HID:(h + 1) * HID], attn_l[h],
                            precision=HI))
    for h in range(NH):
        cols.append(jnp.dot(fc_W[:, h * HID:(h + 1) * HID], attn_r[h],
                            precision=HI))
    A = jnp.stack(cols, axis=1)
    return jnp.concatenate([A, jnp.zeros((HID, HID - 2 * NH), jnp.float32)],
                           axis=1)


def _tc_encode_body(x_ref, encW_ref, encb_ref, fcW_ref, al_ref, ar_ref,
                    he_ref, hx_ref):
    he = jnp.dot(x_ref[...], encW_ref[...], precision=HI) + encb_ref[...]
    he_ref[...] = he
    A = _fold_attn(fcW_ref[...], al_ref[...], ar_ref[...])
    elr = jnp.dot(he, A, precision=HI)
    hx_ref[...] = jnp.concatenate([he, elr], axis=1)


def _tc_encode(x, enc_W, enc_b, fc_W0, attn_l0, attn_r0):
    return pl.pallas_call(
        _tc_encode_body,
        out_shape=(jax.ShapeDtypeStruct((NPAD, HID), jnp.float32),
                   jax.ShapeDtypeStruct((NPAD, GW), jnp.float32)),
    )(x, enc_W, enc_b.reshape(1, HID), fc_W0, attn_l0, attn_r0)


def _tc_post_body(z_ref, hin_ref, fcW_ref, resW_ref, gatb_ref,
                  downW_ref, downb_ref, fcWn_ref, aln_ref, arn_ref,
                  hout_ref, hx_ref, *, with_elr):
    z = z_ref[...]
    hin = hin_ref[...]
    fcW = fcW_ref[...]
    parts = []
    for h in range(NH):
        parts.append(jnp.dot(z[:, h * HID:(h + 1) * HID],
                             fcW[:, h * HID:(h + 1) * HID], precision=HI))
    rst = jnp.concatenate(parts, axis=1)
    rst = rst + jnp.dot(hin, resW_ref[...], precision=HI)
    rst = rst + gatb_ref[...]
    rst = jnp.where(rst >= 0, rst, 0.01 * rst)
    hout = jnp.dot(rst, downW_ref[...], precision=HI) + downb_ref[...]
    hout_ref[...] = hout
    if with_elr:
        A = _fold_attn(fcWn_ref[...], aln_ref[...], arn_ref[...])
        elr = jnp.dot(hout, A, precision=HI)
        hx_ref[...] = jnp.concatenate([hout, elr], axis=1)
    else:
        hx_ref[...] = jnp.zeros_like(hx_ref)


def _tc_post(z, hin, fc_W, res_W, gat_b, down_W, down_b,
             fc_Wn, attn_ln, attn_rn, with_elr):
    blk = 1344
    grid = NPAD // blk
    return pl.pallas_call(
        functools.partial(_tc_post_body, with_elr=with_elr),
        grid=(grid,),
        in_specs=[
            pl.BlockSpec((blk, ZROW), lambda i: (i, 0)),
            pl.BlockSpec((blk, HID), lambda i: (i, 0)),
            pl.BlockSpec((HID, ZROW), lambda i: (0, 0)),
            pl.BlockSpec((HID, ZROW), lambda i: (0, 0)),
            pl.BlockSpec((1, ZROW), lambda i: (0, 0)),
            pl.BlockSpec((ZROW, HID), lambda i: (0, 0)),
            pl.BlockSpec((1, HID), lambda i: (0, 0)),
            pl.BlockSpec((HID, ZROW), lambda i: (0, 0)),
            pl.BlockSpec((NH, HID), lambda i: (0, 0)),
            pl.BlockSpec((NH, HID), lambda i: (0, 0)),
        ],
        out_specs=(pl.BlockSpec((blk, HID), lambda i: (i, 0)),
                   pl.BlockSpec((blk, GW), lambda i: (i, 0))),
        out_shape=(jax.ShapeDtypeStruct((NPAD, HID), jnp.float32),
                   jax.ShapeDtypeStruct((NPAD, GW), jnp.float32)),
    )(z, hin, fc_W, res_W, gat_b.reshape(1, ZROW), down_W,
      down_b.reshape(1, HID), fc_Wn, attn_ln, attn_rn)


def _tc_head_body(h_ref, he_ref, gW_ref, gb_ref, gam_ref, bet_ref,
                  cW_ref, cb_ref, out_ref):
    hg = jnp.concatenate([h_ref[...], he_ref[...]], axis=1)
    mean = jnp.mean(hg, axis=0, keepdims=True)
    var = jnp.mean((hg - mean) ** 2, axis=0, keepdims=True)
    hg = gam_ref[...] * (hg - mean) / jnp.sqrt(var + 1e-5) + bet_ref[...]
    logit = jnp.dot(hg, gW_ref[...], precision=HI) + gb_ref[...]
    m = jnp.max(logit, axis=0, keepdims=True)
    eg = jnp.exp(logit - m)
    gate = eg / jnp.sum(eg, axis=0, keepdims=True)
    pooled = jnp.sum(gate * hg, axis=0, keepdims=True)
    out_ref[...] = jnp.dot(pooled, cW_ref[...], precision=HI) + cb_ref[...]


def _tc_head(h, he, gate_W, gate_b, bn_gamma, bn_beta, cls_W, cls_b):
    nc = cls_W.shape[1]
    return pl.pallas_call(
        _tc_head_body,
        out_shape=jax.ShapeDtypeStruct((1, nc), jnp.float32),
    )(h, he, gate_W, gate_b.reshape(1, 1), bn_gamma.reshape(1, 2 * HID),
      bn_beta.reshape(1, 2 * HID), cls_W, cls_b.reshape(1, nc))


def _er_flat(hx):
    return hx[:, HID:HID + 16].reshape(-1)


def kernel(x, edge_index, enc_W, enc_b, fc_W0, attn_l0, attn_r0, res_W0,
           gat_b0, down_W0, down_b0, fc_W1, attn_l1, attn_r1, res_W1,
           gat_b1, down_W1, down_b1, gate_W, gate_b, bn_gamma, bn_beta,
           cls_W, cls_b):
    src = edge_index[0]
    dst = edge_index[1]
    dst_s, src_s = lax.sort((dst, src), num_keys=1)
    off = jnp.searchsorted(dst_s, jnp.arange(NBK + 1, dtype=jnp.int32) * WB,
                           method="scan_unrolled").astype(jnp.int32)
    off = jnp.concatenate([off, jnp.zeros((240 - NBK - 1,), jnp.int32)])
    pad = jnp.zeros((EPAD - EE,), jnp.int32)
    src_p = jnp.concatenate([src_s, pad])
    dst_p = jnp.concatenate([dst_s, pad])

    xp = jnp.concatenate([x, jnp.zeros((NPAD - NN, DIN), jnp.float32)])
    he, hx0 = _tc_encode(xp, enc_W, enc_b, fc_W0, attn_l0, attn_r0)

    z0 = _sc_edge(src_p, dst_p, off, hx0, _er_flat(hx0))
    z0 = z0.reshape(NPAD, ZROW)
    h1, hx1 = _tc_post(z0, he, fc_W0, res_W0, gat_b0, down_W0, down_b0,
                       fc_W1, attn_l1, attn_r1, with_elr=True)

    z1 = _sc_edge(src_p, dst_p, off, hx1, _er_flat(hx1))
    z1 = z1.reshape(NPAD, ZROW)
    h2, _ = _tc_post(z1, h1, fc_W1, res_W1, gat_b1, down_W1, down_b1,
                     fc_W1, attn_l1, attn_r1, with_elr=False)

    return _tc_head(h2[:NN], he[:NN], gate_W, gate_b, bn_gamma, bn_beta,
                    cls_W, cls_b)


# speedup vs baseline: 22.5363x; 1.0311x over previous
import functools

import jax
import jax.numpy as jnp
from jax import lax
from jax.experimental import pallas as pl
from jax.experimental.pallas import tpu as pltpu
from jax.experimental.pallas import tpu_sc as plsc

NN = 10000
EE = 320000
DIN = 128
HID = 128
NH = 8
WB = 56
NBK = 192
NPAD = NBK * WB
NWORK = 32
ROUNDS = NBK // NWORK
KA = 128
EPAD = EE + 2 * KA
ZROW = NH * HID
ZWORDS = WB * ZROW
GW = 2 * HID
HI = jax.lax.Precision.HIGHEST


def _sc_edge_body(src_hbm, dst_hbm, off_hbm, hx_hbm, erf_hbm, z_hbm,
                  offv, srcA, dstA, srcB, dstB, gbufA, gbufB, erv,
                  eebuf, den, rden, zacc,
                  sem0, sem1, semGA, semGB):
    wid = lax.axis_index("s") * 2 + lax.axis_index("c")
    iota = lax.iota(jnp.int32, 16)

    pltpu.sync_copy(off_hbm, offv)

    def splat(x):
        return jnp.full((16,), x, jnp.int32)

    def read_off(b):
        return jnp.max(plsc.load_gather(offv, [splat(b)]))

    def round_body(r, _carry):
        b = wid + NWORK * r
        base = b * WB
        off_b = read_off(b)
        off_b1 = read_off(b + 1)
        a_start = jnp.bitwise_and(off_b, jnp.int32(-8))
        nsteps = (off_b1 - a_start + (KA - 1)) // KA

        pltpu.sync_copy(
            erf_hbm.at[pl.ds(pl.multiple_of(b * (WB * 16), 8), WB * 16)], erv)

        def zero_z(i, c):
            plsc.store_scatter(zacc, [i * 16 + iota],
                               jnp.zeros((16,), jnp.float32))
            return c
        lax.fori_loop(0, ZWORDS // 16, zero_z, 0, unroll=8)
        for i in range(WB * NH // 16):
            plsc.store_scatter(den, [i * 16 + iota],
                               jnp.zeros((16,), jnp.float32))

        def gbase_of(step):
            return pl.multiple_of(a_start + step * KA, 8)

        def load_sd(step, sbuf, dbuf):
            gb = gbase_of(step)
            c0 = pltpu.async_copy(src_hbm.at[pl.ds(gb, KA)], sbuf, sem0)
            c1 = pltpu.async_copy(dst_hbm.at[pl.ds(gb, KA)], dbuf, sem1)
            c0.wait()
            c1.wait()

        @pl.when(nsteps > 0)
        def _():
            load_sd(0, srcA, dstA)
            pltpu.async_copy(hx_hbm.at[srcA], gbufA, semGA)

        def pass_one(step, sbuf, dbuf, gbuf, semG, sbuf2, dbuf2, gbuf2,
                     semG2):
            gbase = gbase_of(step)

            @pl.when(step + 1 < nsteps)
            def _():
                load_sd(step + 1, sbuf2, dbuf2)
                pltpu.async_copy(hx_hbm.at[sbuf2], gbuf2, semG2)

            pltpu.make_async_copy(hx_hbm.at[sbuf], gbuf, semG).wait()

            for j in range(KA // 16):
                cidx = j * 16 + iota
                gpos = gbase + cidx
                dstv = dbuf[pl.ds(j * 16, 16)]
                valid = jnp.logical_and(gpos >= off_b, gpos < off_b1)
                dloc = jnp.clip(dstv - base, 0, WB - 1)
                prev = dstv.at[jnp.maximum(iota - 1, 0)].get(
                    mode="promise_in_bounds")
                nxt = dstv.at[jnp.minimum(iota + 1, 15)].get(
                    mode="promise_in_bounds")
                start_run = jnp.logical_or(iota == 0, dstv != prev)
                last_run = jnp.logical_or(iota == 15, dstv != nxt)
                run_start = plsc.cummax(jnp.where(start_run, iota, 0))
                prev_idx = jnp.maximum(run_start - 1, 0)
                for h in range(NH):
                    elh = plsc.load_gather(gbuf, [cidx, splat(HID + h)])
                    erh = plsc.load_gather(erv, [dloc * 16 + (8 + h)])
                    e = elh + erh
                    e = jnp.where(e >= 0, e, 0.2 * e)
                    ee = jnp.where(valid, jnp.exp(e), 0.0)
                    plsc.store_scatter(eebuf, [cidx * NH + h], ee)
                    cs = plsc.cumsum(ee)
                    pcs = cs.at[prev_idx].get(mode="promise_in_bounds")
                    tot = cs - jnp.where(run_start > 0, pcs, 0.0)
                    plsc.addupdate_scatter(den, [dloc * NH + h], tot,
                                           mask=last_run)

            def agg_edge(ei, c2):
                dl16 = plsc.load_gather(dbuf, [splat(ei)])
                dl16 = jnp.clip(dl16 - base, 0, WB - 1)
                zb = dl16 * ZROW
                hv = [plsc.load_gather(gbuf, [splat(ei), v * 16 + iota])
                      for v in range(HID // 16)]
                for h in range(NH):
                    a16 = plsc.load_gather(eebuf, [splat(ei * NH + h)])
                    for v in range(HID // 16):
                        plsc.addupdate_scatter(
                            zacc, [zb + h * HID + v * 16 + iota], a16 * hv[v])
                return c2
            lax.fori_loop(0, KA, agg_edge, 0, unroll=4)

        def one_pass(step, c):
            @pl.when(jnp.bitwise_and(step, 1) == 0)
            def _():
                pass_one(step, srcA, dstA, gbufA, semGA,
                         srcB, dstB, gbufB, semGB)

            @pl.when(jnp.bitwise_and(step, 1) == 1)
            def _():
                pass_one(step, srcB, dstB, gbufB, semGB,
                         srcA, dstA, gbufA, semGA)
            return c
        lax.fori_loop(0, nsteps, one_pass, 0)

        for i in range(WB * NH // 16):
            d16 = den[pl.ds(i * 16, 16)]
            plsc.store_scatter(rden, [i * 16 + iota],
                               jnp.where(d16 > 0, 1.0 / d16, 0.0))

        def scale_z(i, c):
            w16 = i * 16 + iota
            rd = plsc.load_gather(rden, [lax.shift_right_logical(w16, 7)])
            z16 = plsc.load_gather(zacc, [w16])
            plsc.store_scatter(zacc, [w16], z16 * rd)
            return c
        lax.fori_loop(0, ZWORDS // 16, scale_z, 0, unroll=8)

        pltpu.sync_copy(zacc,
                        z_hbm.at[pl.ds(pl.multiple_of(b * ZWORDS, 8), ZWORDS)])
        return _carry

    lax.fori_loop(0, ROUNDS, round_body, 0)


def _sc_edge(src_p, dst_p, off, hx, er_flat):
    mesh = plsc.VectorSubcoreMesh(core_axis_name="c", subcore_axis_name="s")
    fn = pl.kernel(
        _sc_edge_body,
        out_type=jax.ShapeDtypeStruct((NPAD * ZROW,), jnp.float32),
        mesh=mesh,
        compiler_params=pltpu.CompilerParams(needs_layout_passes=False),
        scratch_types=[
            pltpu.VMEM((240,), jnp.int32),
            pltpu.VMEM((KA,), jnp.int32),
            pltpu.VMEM((KA,), jnp.int32),
            pltpu.VMEM((KA,), jnp.int32),
            pltpu.VMEM((KA,), jnp.int32),
            pltpu.VMEM((KA, GW), jnp.float32),
            pltpu.VMEM((KA, GW), jnp.float32),
            pltpu.VMEM((WB * 16,), jnp.float32),
            pltpu.VMEM((KA * NH,), jnp.float32),
            pltpu.VMEM((WB * NH,), jnp.float32),
            pltpu.VMEM((WB * NH,), jnp.float32),
            pltpu.VMEM((ZWORDS,), jnp.float32),
            pltpu.SemaphoreType.DMA,
            pltpu.SemaphoreType.DMA,
            pltpu.SemaphoreType.DMA,
            pltpu.SemaphoreType.DMA,
        ],
    )
    return fn(src_p, dst_p, off, hx, er_flat)


def _fold_attn(fc_W, attn_l, attn_r):
    cols = []
    for h in range(NH):
        cols.append(jnp.dot(fc_W[:, h * HID:(h + 1) * HID], attn_l[h],
                            precision=HI))
    for h in range(NH):
        cols.append(jnp.dot(fc_W[:, h * HID:(h + 1) * HID], attn_r[h],
                            precision=HI))
    A = jnp.stack(cols, axis=1)
    return jnp.concatenate([A, jnp.zeros((HID, HID - 2 * NH), jnp.float32)],
                           axis=1)


def _tc_encode_body(x_ref, encW_ref, encb_ref, fcW_ref, al_ref, ar_ref,
                    he_ref, hx_ref):
    he = jnp.dot(x_ref[...], encW_ref[...], precision=HI) + encb_ref[...]
    he_ref[...] = he
    A = _fold_attn(fcW_ref[...], al_ref[...], ar_ref[...])
    elr = jnp.dot(he, A, precision=HI)
    hx_ref[...] = jnp.concatenate([he, elr], axis=1)


def _tc_encode(x, enc_W, enc_b, fc_W0, attn_l0, attn_r0):
    return pl.pallas_call(
        _tc_encode_body,
        out_shape=(jax.ShapeDtypeStruct((NPAD, HID), jnp.float32),
                   jax.ShapeDtypeStruct((NPAD, GW), jnp.float32)),
    )(x, enc_W, enc_b.reshape(1, HID), fc_W0, attn_l0, attn_r0)


def _tc_post_body(z_ref, hin_ref, fcW_ref, resW_ref, gatb_ref,
                  downW_ref, downb_ref, fcWn_ref, aln_ref, arn_ref,
                  hout_ref, hx_ref, *, with_elr):
    z = z_ref[...]
    hin = hin_ref[...]
    fcW = fcW_ref[...]
    parts = []
    for h in range(NH):
        parts.append(jnp.dot(z[:, h * HID:(h + 1) * HID],
                             fcW[:, h * HID:(h + 1) * HID], precision=HI))
    rst = jnp.concatenate(parts, axis=1)
    rst = rst + jnp.dot(hin, resW_ref[...], precision=HI)
    rst = rst + gatb_ref[...]
    rst = jnp.where(rst >= 0, rst, 0.01 * rst)
    hout = jnp.dot(rst, downW_ref[...], precision=HI) + downb_ref[...]
    hout_ref[...] = hout
    if with_elr:
        A = _fold_attn(fcWn_ref[...], aln_ref[...], arn_ref[...])
        elr = jnp.dot(hout, A, precision=HI)
        hx_ref[...] = jnp.concatenate([hout, elr], axis=1)
    else:
        hx_ref[...] = jnp.zeros_like(hx_ref)


def _tc_post(z, hin, fc_W, res_W, gat_b, down_W, down_b,
             fc_Wn, attn_ln, attn_rn, with_elr):
    blk = 1344
    grid = NPAD // blk
    return pl.pallas_call(
        functools.partial(_tc_post_body, with_elr=with_elr),
        grid=(grid,),
        in_specs=[
            pl.BlockSpec((blk, ZROW), lambda i: (i, 0)),
            pl.BlockSpec((blk, HID), lambda i: (i, 0)),
            pl.BlockSpec((HID, ZROW), lambda i: (0, 0)),
            pl.BlockSpec((HID, ZROW), lambda i: (0, 0)),
            pl.BlockSpec((1, ZROW), lambda i: (0, 0)),
            pl.BlockSpec((ZROW, HID), lambda i: (0, 0)),
            pl.BlockSpec((1, HID), lambda i: (0, 0)),
            pl.BlockSpec((HID, ZROW), lambda i: (0, 0)),
            pl.BlockSpec((NH, HID), lambda i: (0, 0)),
            pl.BlockSpec((NH, HID), lambda i: (0, 0)),
        ],
        out_specs=(pl.BlockSpec((blk, HID), lambda i: (i, 0)),
                   pl.BlockSpec((blk, GW), lambda i: (i, 0))),
        out_shape=(jax.ShapeDtypeStruct((NPAD, HID), jnp.float32),
                   jax.ShapeDtypeStruct((NPAD, GW), jnp.float32)),
    )(z, hin, fc_W, res_W, gat_b.reshape(1, ZROW), down_W,
      down_b.reshape(1, HID), fc_Wn, attn_ln, attn_rn)


def _tc_head_body(h_ref, he_ref, gW_ref, gb_ref, gam_ref, bet_ref,
                  cW_ref, cb_ref, out_ref):
    hg = jnp.concatenate([h_ref[...], he_ref[...]], axis=1)
    mean = jnp.mean(hg, axis=0, keepdims=True)
    var = jnp.mean((hg - mean) ** 2, axis=0, keepdims=True)
    hg = gam_ref[...] * (hg - mean) / jnp.sqrt(var + 1e-5) + bet_ref[...]
    logit = jnp.dot(hg, gW_ref[...], precision=HI) + gb_ref[...]
    m = jnp.max(logit, axis=0, keepdims=True)
    eg = jnp.exp(logit - m)
    gate = eg / jnp.sum(eg, axis=0, keepdims=True)
    pooled = jnp.sum(gate * hg, axis=0, keepdims=True)
    out_ref[...] = jnp.dot(pooled, cW_ref[...], precision=HI) + cb_ref[...]


def _tc_head(h, he, gate_W, gate_b, bn_gamma, bn_beta, cls_W, cls_b):
    nc = cls_W.shape[1]
    return pl.pallas_call(
        _tc_head_body,
        out_shape=jax.ShapeDtypeStruct((1, nc), jnp.float32),
    )(h, he, gate_W, gate_b.reshape(1, 1), bn_gamma.reshape(1, 2 * HID),
      bn_beta.reshape(1, 2 * HID), cls_W, cls_b.reshape(1, nc))


def _er_flat(hx):
    return hx[:, HID:HID + 16].reshape(-1)


def kernel(x, edge_index, enc_W, enc_b, fc_W0, attn_l0, attn_r0, res_W0,
           gat_b0, down_W0, down_b0, fc_W1, attn_l1, attn_r1, res_W1,
           gat_b1, down_W1, down_b1, gate_W, gate_b, bn_gamma, bn_beta,
           cls_W, cls_b):
    src = edge_index[0]
    dst = edge_index[1]
    dst_s, src_s = lax.sort((dst, src), num_keys=1)
    off = jnp.searchsorted(dst_s, jnp.arange(NBK + 1, dtype=jnp.int32) * WB,
                           method="scan_unrolled").astype(jnp.int32)
    off = jnp.concatenate([off, jnp.zeros((240 - NBK - 1,), jnp.int32)])
    pad = jnp.zeros((EPAD - EE,), jnp.int32)
    src_p = jnp.concatenate([src_s, pad])
    dst_p = jnp.concatenate([dst_s, pad])

    xp = jnp.concatenate([x, jnp.zeros((NPAD - NN, DIN), jnp.float32)])
    he, hx0 = _tc_encode(xp, enc_W, enc_b, fc_W0, attn_l0, attn_r0)

    z0 = _sc_edge(src_p, dst_p, off, hx0, _er_flat(hx0))
    z0 = z0.reshape(NPAD, ZROW)
    h1, hx1 = _tc_post(z0, he, fc_W0, res_W0, gat_b0, down_W0, down_b0,
                       fc_W1, attn_l1, attn_r1, with_elr=True)

    z1 = _sc_edge(src_p, dst_p, off, hx1, _er_flat(hx1))
    z1 = z1.reshape(NPAD, ZROW)
    h2, _ = _tc_post(z1, h1, fc_W1, res_W1, gat_b1, down_W1, down_b1,
                     fc_W1, attn_l1, attn_r1, with_elr=False)

    return _tc_head(h2[:NN], he[:NN], gate_W, gate_b, bn_gamma, bn_beta,
                    cls_W, cls_b)
